# baseline (speedup 1.0000x reference)
import jax
import jax.numpy as jnp
from jax.experimental import pallas as pl
from jax.experimental.pallas import tpu as pltpu

T = 1024
I = 100
U = 100000
P = 64

BU = 2048
NU = (U + BU - 1) // BU
TAIL_START = (NU - 1) * BU
TAIL = U - TAIL_START

BT2 = 256


NSTREAM = 8
RS = T // NSTREAM


def _matmul_kernel(oh_hbm, oh_tail_hbm, coef_ref, coef_tail_ref, cu_ref, buf, sem):
    j = pl.program_id(0)
    slot = jax.lax.rem(j, 2)

    def start_j(jj, s):
        @pl.when(jj < NU - 1)
        def _main():
            for k in range(NSTREAM):
                pltpu.make_async_copy(
                    oh_hbm.at[pl.ds(k * RS, RS), 0, pl.ds(jj * BU, BU)],
                    buf.at[s, pl.ds(k * RS, RS)],
                    sem.at[s, k],
                ).start()

        @pl.when(jj == NU - 1)
        def _tail():
            for k in range(NSTREAM):
                pltpu.make_async_copy(
                    oh_tail_hbm.at[pl.ds(k * RS, RS), 0, :],
                    buf.at[s, pl.ds(k * RS, RS)],
                    sem.at[s, k],
                ).start()

    def wait_s(s):
        for k in range(NSTREAM):
            pltpu.make_async_copy(
                oh_hbm.at[pl.ds(0, RS), 0, pl.ds(0, BU)],
                buf.at[s, pl.ds(0, RS)],
                sem.at[s, k],
            ).wait()

    @pl.when(j == 0)
    def _init():
        cu_ref[...] = jnp.zeros_like(cu_ref)
        start_j(0, 0)

    @pl.when(j + 1 < NU)
    def _prefetch():
        start_j(j + 1, 1 - slot)

    wait_s(slot)
    cf = jnp.where(j < NU - 1, coef_ref[...], coef_tail_ref[...])
    cu_ref[...] += jnp.dot(buf[slot], cf, preferred_element_type=jnp.float32)


def _epilogue_kernel(x_ref, cu_ref, out_ref):
    out_ref[...] = jnp.sum(x_ref[...] * cu_ref[...][:, None, :], axis=-1)


@jax.jit
def kernel(x, user_onehot, coef):
    oh_tail = jnp.pad(
        user_onehot[:, :, TAIL_START:], ((0, 0), (0, 0), (0, BU - TAIL))
    )
    coef_tail = jnp.pad(coef[TAIL_START:], ((0, BU - TAIL), (0, 0)))

    cu = pl.pallas_call(
        _matmul_kernel,
        grid=(NU,),
        in_specs=[
            pl.BlockSpec(memory_space=pl.ANY),
            pl.BlockSpec(memory_space=pl.ANY),
            pl.BlockSpec((BU, P), lambda j: (jnp.minimum(j, NU - 2), 0)),
            pl.BlockSpec((BU, P), lambda j: (0, 0)),
        ],
        out_specs=pl.BlockSpec((T, P), lambda j: (0, 0)),
        out_shape=jax.ShapeDtypeStruct((T, P), jnp.float32),
        scratch_shapes=[
            pltpu.VMEM((2, T, BU), jnp.float32),
            pltpu.SemaphoreType.DMA((2, NSTREAM)),
        ],
        compiler_params=pltpu.CompilerParams(
            dimension_semantics=("arbitrary",),
        ),
    )(user_onehot, oh_tail, coef, coef_tail)

    out = pl.pallas_call(
        _epilogue_kernel,
        grid=(T // BT2,),
        in_specs=[
            pl.BlockSpec((BT2, I, P), lambda i: (i, 0, 0)),
            pl.BlockSpec((BT2, P), lambda i: (i, 0)),
        ],
        out_specs=pl.BlockSpec((BT2, I), lambda i: (i, 0)),
        out_shape=jax.ShapeDtypeStruct((T, I), jnp.float32),
        compiler_params=pltpu.CompilerParams(
            dimension_semantics=("parallel",),
        ),
    )(x, cu)

    return out


# speedup vs baseline: 1.0144x; 1.0144x over previous
import jax
import jax.numpy as jnp
from jax.experimental import pallas as pl
from jax.experimental.pallas import tpu as pltpu

T = 1024
I = 100
U = 100000
P = 64

BU = 2048
NU = (U + BU - 1) // BU

BT2 = 256


def _matmul_kernel(oh_ref, coef_ref, cu_ref):
    j = pl.program_id(0)

    @pl.when(j == 0)
    def _init():
        cu_ref[...] = jnp.zeros_like(cu_ref)

    oh = oh_ref[...]
    cf = coef_ref[...]
    limit = U - j * BU
    col = jax.lax.broadcasted_iota(jnp.int32, oh.shape, 1)
    oh = jnp.where(col < limit, oh, 0.0)
    row = jax.lax.broadcasted_iota(jnp.int32, cf.shape, 0)
    cf = jnp.where(row < limit, cf, 0.0)
    cu_ref[...] += jnp.dot(oh, cf, preferred_element_type=jnp.float32)


def _epilogue_kernel(x_ref, cu_ref, out_ref):
    out_ref[...] = jnp.sum(x_ref[...] * cu_ref[...][:, None, :], axis=-1)


def _kernel_impl(x, user_onehot, coef):
    cu = pl.pallas_call(
        _matmul_kernel,
        grid=(NU,),
        in_specs=[
            pl.BlockSpec((T, None, BU), lambda j: (0, 0, j)),
            pl.BlockSpec((BU, P), lambda j: (j, 0)),
        ],
        out_specs=pl.BlockSpec((T, P), lambda j: (0, 0)),
        out_shape=jax.ShapeDtypeStruct((T, P), jnp.float32),
        compiler_params=pltpu.CompilerParams(
            dimension_semantics=("arbitrary",),
        ),
    )(user_onehot, coef)

    out = pl.pallas_call(
        _epilogue_kernel,
        grid=(T // BT2,),
        in_specs=[
            pl.BlockSpec((BT2, I, P), lambda i: (i, 0, 0)),
            pl.BlockSpec((BT2, P), lambda i: (i, 0)),
        ],
        out_specs=pl.BlockSpec((BT2, I), lambda i: (i, 0)),
        out_shape=jax.ShapeDtypeStruct((T, I), jnp.float32),
        compiler_params=pltpu.CompilerParams(
            dimension_semantics=("parallel",),
        ),
    )(x, cu)

    return out


kernel = jax.jit(_kernel_impl)
